# baseline (speedup 1.0000x reference)
import jax
import jax.numpy as jnp
from jax.experimental import pallas as pl
from jax.experimental.pallas import tpu as pltpu

_RC = 1024
_NS = 4


def _body(hs_hbm, tv_hbm, idx_hbm, wd_hbm, wu_hbm, out_hbm,
          wd_v, wu_v, hs_v, tv_v, idx_v, out_v, in_sems, w_sem, out_sems):
    g = pl.program_id(0)
    S = pl.num_programs(0)
    s = g % _NS

    def start_in(chunk, slot):
        pltpu.make_async_copy(
            hs_hbm.at[pl.ds(chunk * _RC, _RC)], hs_v.at[slot],
            in_sems.at[slot]).start()
        pltpu.make_async_copy(
            tv_hbm.at[pl.ds(chunk * _RC, _RC)], tv_v.at[slot],
            in_sems.at[slot]).start()
        pltpu.make_async_copy(
            idx_hbm.at[pl.ds(chunk * _RC, _RC)], idx_v.at[slot],
            in_sems.at[slot]).start()

    def wait_in(slot):
        pltpu.make_async_copy(
            hs_hbm.at[pl.ds(0, _RC)], hs_v.at[slot], in_sems.at[slot]).wait()
        pltpu.make_async_copy(
            tv_hbm.at[pl.ds(0, _RC)], tv_v.at[slot], in_sems.at[slot]).wait()
        pltpu.make_async_copy(
            idx_hbm.at[pl.ds(0, _RC)], idx_v.at[slot], in_sems.at[slot]).wait()

    def wait_out(chunk, slot):
        pltpu.make_async_copy(
            out_v.at[slot], out_hbm.at[pl.ds(chunk * _RC, _RC)],
            out_sems.at[slot]).wait()

    @pl.when(g == 0)
    def _prologue():
        pltpu.make_async_copy(wd_hbm, wd_v, w_sem).start()
        pltpu.make_async_copy(wu_hbm, wu_v, w_sem).start()
        for c in range(_NS - 1):
            start_in(c, c)
        pltpu.make_async_copy(wd_hbm, wd_v, w_sem).wait()
        pltpu.make_async_copy(wu_hbm, wu_v, w_sem).wait()

    wait_in(s)

    @pl.when(g + _NS - 1 < S)
    def _next_in():
        start_in(g + _NS - 1, (g + _NS - 1) % _NS)

    @pl.when(g >= _NS)
    def _recycle_out():
        wait_out(g - _NS, s)

    hs = hs_v[s]
    rank = wd_v.shape[0]
    down = jax.lax.dot_general(
        hs, wd_v[...], (((1,), (1,)), ((), ())),
        preferred_element_type=jnp.float32)
    iota = jax.lax.broadcasted_iota(jnp.int32, (_RC, rank), 1)
    idx = idx_v[s]
    tv = tv_v[s]
    w = jnp.zeros((_RC, rank), jnp.float32)
    for k in range(idx.shape[1]):
        w = jnp.where(iota == idx[:, k:k + 1], tv[:, k:k + 1], w)
    out_v[s] = jax.lax.dot_general(
        down * w, wu_v[...], (((1,), (1,)), ((), ())),
        preferred_element_type=jnp.float32)

    pltpu.make_async_copy(
        out_v.at[s], out_hbm.at[pl.ds(g * _RC, _RC)], out_sems.at[s]).start()

    @pl.when(g == S - 1)
    def _epilogue():
        for d in range(_NS - 1, 0, -1):
            wait_out(g - d, (g - d) % _NS)
        wait_out(g, s)


def kernel(hidden_states, top_k_values, top_k_indices, W_down, W_up):
    N, d_in = hidden_states.shape
    rank, _ = W_down.shape
    d_out, _ = W_up.shape
    top_k = top_k_values.shape[1]
    S = N // _RC
    any_spec = pl.BlockSpec(memory_space=pltpu.MemorySpace.HBM)
    return pl.pallas_call(
        _body,
        grid=(S,),
        in_specs=[any_spec] * 5,
        out_specs=any_spec,
        out_shape=jax.ShapeDtypeStruct((N, d_out), jnp.float32),
        scratch_shapes=[
            pltpu.VMEM((rank, d_in), jnp.float32),
            pltpu.VMEM((d_out, rank), jnp.float32),
            pltpu.VMEM((_NS, _RC, d_in), jnp.float32),
            pltpu.VMEM((_NS, _RC, top_k), jnp.float32),
            pltpu.VMEM((_NS, _RC, top_k), jnp.int32),
            pltpu.VMEM((_NS, _RC, d_out), jnp.float32),
            pltpu.SemaphoreType.DMA((_NS,)),
            pltpu.SemaphoreType.DMA,
            pltpu.SemaphoreType.DMA((_NS,)),
        ],
        compiler_params=pltpu.CompilerParams(
            dimension_semantics=("arbitrary",),
        ),
    )(hidden_states, top_k_values, top_k_indices.astype(jnp.int32),
      W_down, W_up)


# speedup vs baseline: 1.0158x; 1.0158x over previous
import jax
import jax.numpy as jnp
from jax.experimental import pallas as pl
from jax.experimental.pallas import tpu as pltpu

_RC = 1024
_NS = 3


def _body(hs_hbm, tv_hbm, idx_hbm, wd_hbm, wu_hbm, out_hbm,
          wd_v, wu_v, hs_v, tv_v, idx_v, out_v, in_sems, w_sem, out_sems):
    g = pl.program_id(0)
    S = pl.num_programs(0)
    s = g % _NS

    def start_in(chunk, slot):
        pltpu.make_async_copy(
            hs_hbm.at[pl.ds(chunk * _RC, _RC)], hs_v.at[slot],
            in_sems.at[slot]).start()
        pltpu.make_async_copy(
            tv_hbm.at[pl.ds(chunk * _RC, _RC)], tv_v.at[slot],
            in_sems.at[slot]).start()
        pltpu.make_async_copy(
            idx_hbm.at[pl.ds(chunk * _RC, _RC)], idx_v.at[slot],
            in_sems.at[slot]).start()

    def wait_in(slot):
        pltpu.make_async_copy(
            hs_hbm.at[pl.ds(0, _RC)], hs_v.at[slot], in_sems.at[slot]).wait()
        pltpu.make_async_copy(
            tv_hbm.at[pl.ds(0, _RC)], tv_v.at[slot], in_sems.at[slot]).wait()
        pltpu.make_async_copy(
            idx_hbm.at[pl.ds(0, _RC)], idx_v.at[slot], in_sems.at[slot]).wait()

    def wait_out(chunk, slot):
        pltpu.make_async_copy(
            out_v.at[slot], out_hbm.at[pl.ds(chunk * _RC, _RC)],
            out_sems.at[slot]).wait()

    @pl.when(g == 0)
    def _prologue():
        pltpu.make_async_copy(wd_hbm, wd_v, w_sem).start()
        pltpu.make_async_copy(wu_hbm, wu_v, w_sem).start()
        for c in range(_NS - 1):
            start_in(c, c)
        pltpu.make_async_copy(wd_hbm, wd_v, w_sem).wait()
        pltpu.make_async_copy(wu_hbm, wu_v, w_sem).wait()

    wait_in(s)

    @pl.when(g + _NS - 1 < S)
    def _next_in():
        start_in(g + _NS - 1, (g + _NS - 1) % _NS)

    @pl.when(g >= _NS)
    def _recycle_out():
        wait_out(g - _NS, s)

    def compute(lo, rows):
        hs = hs_v[s, pl.ds(lo, rows)]
        rank = wd_v.shape[0]
        down = jax.lax.dot_general(
            hs, wd_v[...], (((1,), (1,)), ((), ())),
            preferred_element_type=jnp.float32)
        iota = jax.lax.broadcasted_iota(jnp.int32, (rows, rank), 1)
        idx = idx_v[s, pl.ds(lo, rows)]
        tv = tv_v[s, pl.ds(lo, rows)]
        w = jnp.zeros((rows, rank), jnp.float32)
        for k in range(idx.shape[1]):
            w = jnp.where(iota == idx[:, k:k + 1], tv[:, k:k + 1], w)
        out_v[s, pl.ds(lo, rows)] = jax.lax.dot_general(
            down * w, wu_v[...], (((1,), (1,)), ((), ())),
            preferred_element_type=jnp.float32)
        pltpu.make_async_copy(
            out_v.at[s, pl.ds(lo, rows)],
            out_hbm.at[pl.ds(g * _RC + lo, rows)], out_sems.at[s]).start()

    @pl.when(g < S - 1)
    def _full_chunk():
        compute(0, _RC)

    @pl.when(g == S - 1)
    def _split_chunk():
        compute(0, _RC // 2)
        compute(_RC // 2, _RC // 2)

    @pl.when(g == S - 1)
    def _epilogue():
        for d in range(_NS - 1, 0, -1):
            wait_out(g - d, (g - d) % _NS)
        wait_out(g, s)


def kernel(hidden_states, top_k_values, top_k_indices, W_down, W_up):
    N, d_in = hidden_states.shape
    rank, _ = W_down.shape
    d_out, _ = W_up.shape
    top_k = top_k_values.shape[1]
    S = N // _RC
    any_spec = pl.BlockSpec(memory_space=pltpu.MemorySpace.HBM)
    return pl.pallas_call(
        _body,
        grid=(S,),
        in_specs=[any_spec] * 5,
        out_specs=any_spec,
        out_shape=jax.ShapeDtypeStruct((N, d_out), jnp.float32),
        scratch_shapes=[
            pltpu.VMEM((rank, d_in), jnp.float32),
            pltpu.VMEM((d_out, rank), jnp.float32),
            pltpu.VMEM((_NS, _RC, d_in), jnp.float32),
            pltpu.VMEM((_NS, _RC, top_k), jnp.float32),
            pltpu.VMEM((_NS, _RC, top_k), jnp.int32),
            pltpu.VMEM((_NS, _RC, d_out), jnp.float32),
            pltpu.SemaphoreType.DMA((_NS,)),
            pltpu.SemaphoreType.DMA,
            pltpu.SemaphoreType.DMA((_NS,)),
        ],
        compiler_params=pltpu.CompilerParams(
            dimension_semantics=("arbitrary",),
        ),
    )(hidden_states, top_k_values, top_k_indices.astype(jnp.int32),
      W_down, W_up)


# speedup vs baseline: 1.0209x; 1.0049x over previous
import jax
import jax.numpy as jnp
from jax.experimental import pallas as pl
from jax.experimental.pallas import tpu as pltpu

_RC = 1024
_NS = 3


def _body(hs_hbm, tv_hbm, idx_hbm, wd_hbm, wu_hbm, out_hbm,
          wd_v, wu_v, hs_v, tv_v, idx_v, out_v, in_sems, w_sem, out_sems,
          ramp_sem):
    g = pl.program_id(0)
    S = pl.num_programs(0)
    s = g % _NS

    def start_in(chunk, slot):
        pltpu.make_async_copy(
            hs_hbm.at[pl.ds(chunk * _RC, _RC)], hs_v.at[slot],
            in_sems.at[slot]).start()
        pltpu.make_async_copy(
            tv_hbm.at[pl.ds(chunk * _RC, _RC)], tv_v.at[slot],
            in_sems.at[slot]).start()
        pltpu.make_async_copy(
            idx_hbm.at[pl.ds(chunk * _RC, _RC)], idx_v.at[slot],
            in_sems.at[slot]).start()

    def wait_in(slot):
        pltpu.make_async_copy(
            hs_hbm.at[pl.ds(0, _RC)], hs_v.at[slot], in_sems.at[slot]).wait()
        pltpu.make_async_copy(
            tv_hbm.at[pl.ds(0, _RC)], tv_v.at[slot], in_sems.at[slot]).wait()
        pltpu.make_async_copy(
            idx_hbm.at[pl.ds(0, _RC)], idx_v.at[slot], in_sems.at[slot]).wait()

    def wait_out(chunk, slot):
        pltpu.make_async_copy(
            out_v.at[slot], out_hbm.at[pl.ds(chunk * _RC, _RC)],
            out_sems.at[slot]).wait()

    h = _RC // 2

    @pl.when(g == 0)
    def _prologue():
        pltpu.make_async_copy(wd_hbm, wd_v, w_sem).start()
        pltpu.make_async_copy(wu_hbm, wu_v, w_sem).start()
        pltpu.make_async_copy(
            hs_hbm.at[pl.ds(0, h)], hs_v.at[0, pl.ds(0, h)],
            in_sems.at[0]).start()
        pltpu.make_async_copy(
            tv_hbm.at[pl.ds(0, _RC)], tv_v.at[0], in_sems.at[0]).start()
        pltpu.make_async_copy(
            idx_hbm.at[pl.ds(0, _RC)], idx_v.at[0], in_sems.at[0]).start()
        pltpu.make_async_copy(
            hs_hbm.at[pl.ds(h, h)], hs_v.at[0, pl.ds(h, h)],
            ramp_sem).start()
        start_in(1, 1)
        pltpu.make_async_copy(wd_hbm, wd_v, w_sem).wait()
        pltpu.make_async_copy(wu_hbm, wu_v, w_sem).wait()

    @pl.when(g > 0)
    def _wait_full():
        wait_in(s)

    @pl.when(g + _NS - 1 < S)
    def _next_in():
        start_in(g + _NS - 1, (g + _NS - 1) % _NS)

    @pl.when(g >= _NS)
    def _recycle_out():
        wait_out(g - _NS, s)

    def compute(lo, rows):
        hs = hs_v[s, pl.ds(lo, rows)]
        rank = wd_v.shape[0]
        down = jax.lax.dot_general(
            hs, wd_v[...], (((1,), (1,)), ((), ())),
            preferred_element_type=jnp.float32)
        iota = jax.lax.broadcasted_iota(jnp.int32, (rows, rank), 1)
        idx = idx_v[s, pl.ds(lo, rows)]
        tv = tv_v[s, pl.ds(lo, rows)]
        w = jnp.zeros((rows, rank), jnp.float32)
        for k in range(idx.shape[1]):
            w = jnp.where(iota == idx[:, k:k + 1], tv[:, k:k + 1], w)
        out_v[s, pl.ds(lo, rows)] = jax.lax.dot_general(
            down * w, wu_v[...], (((1,), (1,)), ((), ())),
            preferred_element_type=jnp.float32)
        pltpu.make_async_copy(
            out_v.at[s, pl.ds(lo, rows)],
            out_hbm.at[pl.ds(g * _RC + lo, rows)], out_sems.at[s]).start()

    @pl.when(g == 0)
    def _first_chunk():
        pltpu.make_async_copy(
            hs_hbm.at[pl.ds(0, h)], hs_v.at[0, pl.ds(0, h)],
            in_sems.at[0]).wait()
        pltpu.make_async_copy(
            tv_hbm.at[pl.ds(0, _RC)], tv_v.at[0], in_sems.at[0]).wait()
        pltpu.make_async_copy(
            idx_hbm.at[pl.ds(0, _RC)], idx_v.at[0], in_sems.at[0]).wait()
        compute(0, h)
        pltpu.make_async_copy(
            hs_hbm.at[pl.ds(h, h)], hs_v.at[0, pl.ds(h, h)],
            ramp_sem).wait()
        compute(h, h)

    @pl.when(jnp.logical_and(g > 0, g < S - 1))
    def _full_chunk():
        compute(0, _RC)

    @pl.when(g == S - 1)
    def _split_chunk():
        compute(0, h)
        compute(h, h)

    @pl.when(g == S - 1)
    def _epilogue():
        for d in range(_NS - 1, 0, -1):
            wait_out(g - d, (g - d) % _NS)
        wait_out(g, s)


def kernel(hidden_states, top_k_values, top_k_indices, W_down, W_up):
    N, d_in = hidden_states.shape
    rank, _ = W_down.shape
    d_out, _ = W_up.shape
    top_k = top_k_values.shape[1]
    S = N // _RC
    any_spec = pl.BlockSpec(memory_space=pltpu.MemorySpace.HBM)
    return pl.pallas_call(
        _body,
        grid=(S,),
        in_specs=[any_spec] * 5,
        out_specs=any_spec,
        out_shape=jax.ShapeDtypeStruct((N, d_out), jnp.float32),
        scratch_shapes=[
            pltpu.VMEM((rank, d_in), jnp.float32),
            pltpu.VMEM((d_out, rank), jnp.float32),
            pltpu.VMEM((_NS, _RC, d_in), jnp.float32),
            pltpu.VMEM((_NS, _RC, top_k), jnp.float32),
            pltpu.VMEM((_NS, _RC, top_k), jnp.int32),
            pltpu.VMEM((_NS, _RC, d_out), jnp.float32),
            pltpu.SemaphoreType.DMA((_NS,)),
            pltpu.SemaphoreType.DMA,
            pltpu.SemaphoreType.DMA((_NS,)),
            pltpu.SemaphoreType.DMA,
        ],
        compiler_params=pltpu.CompilerParams(
            dimension_semantics=("arbitrary",),
        ),
    )(hidden_states, top_k_values, top_k_indices.astype(jnp.int32),
      W_down, W_up)
